# baseline (speedup 1.0000x reference)
import jax
import jax.numpy as jnp
from jax.experimental import pallas as pl

N = 10000
E = 320000
DL = 16
DH = 128
NLAYERS = 3

_BLK = 1000


def _gates_body(xin_ref, xout_ref, hin_ref, hout_ref, c_ref, Wg_ref, bg_ref,
                h_ref, cout_ref):
    inp = jnp.concatenate(
        [xin_ref[...], xout_ref[...], hin_ref[...], hout_ref[...]], axis=1)
    g = jnp.dot(inp, Wg_ref[...], preferred_element_type=jnp.float32)
    g = g + bg_ref[...]
    gi = jax.nn.sigmoid(g[:, 0 * DH:1 * DH])
    go = jax.nn.sigmoid(g[:, 1 * DH:2 * DH])
    gf = jax.nn.sigmoid(g[:, 2 * DH:3 * DH])
    gu = jnp.tanh(g[:, 3 * DH:4 * DH])
    c = gf * c_ref[...] + gi * gu
    cout_ref[...] = c
    h_ref[...] = go * jnp.tanh(c)


def _gates(x_in, x_out, h_in, h_out, c, Wg, bg):
    row = pl.BlockSpec((_BLK, DH), lambda i: (i, 0))
    full_w = pl.BlockSpec((4 * DH, 4 * DH), lambda i: (0, 0))
    full_b = pl.BlockSpec((1, 4 * DH), lambda i: (0, 0))
    return pl.pallas_call(
        _gates_body,
        grid=(N // _BLK,),
        in_specs=[row, row, row, row, row, full_w, full_b],
        out_specs=[row, row],
        out_shape=[jax.ShapeDtypeStruct((N, DH), jnp.float32),
                   jax.ShapeDtypeStruct((N, DH), jnp.float32)],
    )(x_in, x_out, h_in, h_out, c, Wg, bg)


def kernel(i_token, i_link, i_from, i_to, emb_token, emb_link, W_c, b_c,
           W_l, b_l, W_gi, b_gi, W_go, b_go, W_gf, b_gf, W_gu, b_gu):
    wemb = jnp.take(emb_token, i_token, axis=0)
    node_emb = jnp.tanh(wemb @ W_c + b_c)
    nodeL = node_emb @ W_l[DL:]
    labelL = emb_link @ W_l[:DL] + b_l

    u = jnp.tanh(jnp.take(labelL, i_link, axis=0) + jnp.take(nodeL, i_from, axis=0))
    x_in = jax.ops.segment_sum(u, i_to, num_segments=N)
    x_out = jax.ops.segment_sum(u, i_from, num_segments=N)

    Wg = jnp.concatenate([W_gi, W_go, W_gf, W_gu], axis=1)
    bg = jnp.concatenate([b_gi, b_go, b_gf, b_gu]).reshape(1, 4 * DH)

    h = jnp.zeros((N, DH), dtype=jnp.float32)
    c = jnp.zeros((N, DH), dtype=jnp.float32)
    for _ in range(NLAYERS):
        h_in = jax.ops.segment_sum(jnp.take(h, i_from, axis=0), i_to, num_segments=N)
        h_out = jax.ops.segment_sum(jnp.take(h, i_to, axis=0), i_from, num_segments=N)
        h, c = _gates(x_in, x_out, h_in, h_out, c, Wg, bg)
    return h


# speedup vs baseline: 5.6461x; 5.6461x over previous
import functools

import jax
import jax.numpy as jnp
from jax import lax
from jax.experimental import pallas as pl
from jax.experimental.pallas import tpu as pltpu
from jax.experimental.pallas import tpu_sc as plsc

N = 10000
NP = 10240
E = 320000
DL = 16
DW = 128
DH = 128
NLAYERS = 3
NLBL_PAD = 64

NC, NS = 2, 16
NW = NC * NS
CH = 80

EW = E // NW
CW = EW // CH
ET = E // NS
CT = ET // CH
BI = 25
NT = NP // NS

_MESH = plsc.VectorSubcoreMesh(core_axis_name="c", subcore_axis_name="s")


@functools.partial(
    pl.kernel,
    out_type=jax.ShapeDtypeStruct((NP, DW), jnp.float32),
    mesh=_MESH,
    scratch_types=[
        pltpu.VMEM((NP // NW // CH, CH), jnp.int32),
        pltpu.VMEM((CH, DW), jnp.float32),
        pltpu.SemaphoreType.DMA,
    ],
)
def _tok_gather(tok3, emb_token, out, idx_v, buf, sem):
    wid = lax.axis_index("c") * NS + lax.axis_index("s")
    pltpu.sync_copy(tok3.at[wid], idx_v)
    nch = NP // NW // CH

    def body(j, _):
        pltpu.async_copy(emb_token.at[idx_v.at[j]], buf, sem).wait()
        pltpu.sync_copy(buf, out.at[pl.ds(wid * (NP // NW) + j * CH, CH)])
        return _

    lax.fori_loop(0, nch, body, None)


@functools.partial(
    pl.kernel,
    out_type=jax.ShapeDtypeStruct((E, DH), jnp.float32),
    mesh=_MESH,
    scratch_types=[
        pltpu.VMEM((CW, CH), jnp.int32),
        pltpu.VMEM((CH, DH), jnp.float32),
        pltpu.SemaphoreType.DMA,
    ],
)
def _edge_gather(frm3, nodeL, s_out, fidx, buf, sem):
    wid = lax.axis_index("c") * NS + lax.axis_index("s")
    pltpu.sync_copy(frm3.at[wid], fidx)

    def body(j, _):
        pltpu.async_copy(nodeL.at[fidx.at[j]], buf, sem).wait()
        pltpu.sync_copy(buf, s_out.at[pl.ds(wid * EW + j * CH, CH)])
        return _

    lax.fori_loop(0, CW, body, None)


@functools.partial(
    pl.kernel,
    out_type=jax.ShapeDtypeStruct((NC, NP, DH), jnp.float32),
    mesh=_MESH,
    scratch_types=[
        pltpu.VMEM((CT, CH), jnp.int32),
        pltpu.VMEM((CH, DH), jnp.float32),
        pltpu.VMEM_SHARED((NP, DH), jnp.float32),
        pltpu.SemaphoreType.DMA,
    ],
)
def _x_sums(u, sidx4, zeros, out, sidx, buf, acc, sem):
    c = lax.axis_index("c")
    s = lax.axis_index("s")
    pltpu.sync_copy(sidx4.at[c, s], sidx)
    pltpu.sync_copy(zeros.at[pl.ds(s * NT, NT)], acc.at[pl.ds(s * NT, NT)])
    plsc.subcore_barrier()

    def body(j, _):
        pltpu.sync_copy(u.at[pl.ds(s * ET + j * CH, CH)], buf)
        pltpu.sync_copy(buf, acc.at[sidx.at[j]], add=True)
        return _

    lax.fori_loop(0, CT, body, None)
    plsc.subcore_barrier()
    pltpu.sync_copy(acc.at[pl.ds(s * NT, NT)], out.at[c, pl.ds(s * NT, NT)])


@functools.partial(
    pl.kernel,
    out_type=jax.ShapeDtypeStruct((NC, NP, DH), jnp.float32),
    mesh=_MESH,
    scratch_types=[
        pltpu.VMEM((BI, CH), jnp.int32),
        pltpu.VMEM((BI, CH), jnp.int32),
        pltpu.VMEM((CH, DH), jnp.float32),
        pltpu.VMEM_SHARED((NP, DH), jnp.float32),
        pltpu.SemaphoreType.DMA,
    ],
)
def _h_sums(h, gidx4, sidx4, zeros, out, gidx, sidx, buf, acc, sem):
    c = lax.axis_index("c")
    s = lax.axis_index("s")
    pltpu.sync_copy(zeros.at[pl.ds(s * NT, NT)], acc.at[pl.ds(s * NT, NT)])
    plsc.subcore_barrier()

    def blk(jb, _):
        pltpu.sync_copy(gidx4.at[c, s, jb], gidx)
        pltpu.sync_copy(sidx4.at[c, s, jb], sidx)

        def body(j, __):
            pltpu.async_copy(h.at[gidx.at[j]], buf, sem).wait()
            pltpu.sync_copy(buf, acc.at[sidx.at[j]], add=True)
            return __

        lax.fori_loop(0, BI, body, None)
        return _

    lax.fori_loop(0, CT // BI, blk, None)
    plsc.subcore_barrier()
    pltpu.sync_copy(acc.at[pl.ds(s * NT, NT)], out.at[c, pl.ds(s * NT, NT)])


_NBLK = 640
_EBLK = 2000


def _tables_body(wemb_ref, Wc_ref, bc_ref, Wl1_ref, Wl2_ref, bl_ref, embl_ref,
                 nodeL_ref, labelL_ref):
    t = jnp.tanh(jnp.dot(wemb_ref[...], Wc_ref[...],
                         preferred_element_type=jnp.float32) + bc_ref[...])
    nodeL_ref[...] = jnp.dot(t, Wl2_ref[...], preferred_element_type=jnp.float32)
    labelL_ref[...] = jnp.dot(embl_ref[...], Wl1_ref[...],
                              preferred_element_type=jnp.float32) + bl_ref[...]


def _tables(wemb, Wc, bc, Wl1, Wl2, bl, embl):
    return pl.pallas_call(
        _tables_body,
        grid=(NP // _NBLK,),
        in_specs=[
            pl.BlockSpec((_NBLK, DW), lambda i: (i, 0)),
            pl.BlockSpec((DW, DH), lambda i: (0, 0)),
            pl.BlockSpec((1, DH), lambda i: (0, 0)),
            pl.BlockSpec((DW, DH), lambda i: (0, 0)),
            pl.BlockSpec((DW, DH), lambda i: (0, 0)),
            pl.BlockSpec((1, DH), lambda i: (0, 0)),
            pl.BlockSpec((NLBL_PAD, DW), lambda i: (0, 0)),
        ],
        out_specs=[
            pl.BlockSpec((_NBLK, DH), lambda i: (i, 0)),
            pl.BlockSpec((NLBL_PAD, DH), lambda i: (0, 0)),
        ],
        out_shape=[jax.ShapeDtypeStruct((NP, DH), jnp.float32),
                   jax.ShapeDtypeStruct((NLBL_PAD, DH), jnp.float32)],
    )(wemb, Wc, bc, Wl1, Wl2, bl, embl)


def _edge_tanh_body(s_ref, lnk_ref, labelL_ref, u_ref):
    lnk = lnk_ref[...].reshape(_EBLK, 1)
    onehot = (lnk == lax.broadcasted_iota(jnp.int32, (1, NLBL_PAD), 1))
    lbl = jnp.dot(onehot.astype(jnp.float32), labelL_ref[...],
                  preferred_element_type=jnp.float32)
    u_ref[...] = jnp.tanh(s_ref[...] + lbl)


def _edge_tanh(s, lnk3, labelL):
    return pl.pallas_call(
        _edge_tanh_body,
        grid=(E // _EBLK,),
        in_specs=[
            pl.BlockSpec((_EBLK, DH), lambda i: (i, 0)),
            pl.BlockSpec((1, 1, _EBLK), lambda i: (i, 0, 0)),
            pl.BlockSpec((NLBL_PAD, DH), lambda i: (0, 0)),
        ],
        out_specs=pl.BlockSpec((_EBLK, DH), lambda i: (i, 0)),
        out_shape=jax.ShapeDtypeStruct((E, DH), jnp.float32),
    )(s, lnk3, labelL)


def _gates_body(xin_ref, xout_ref, hin_ref, hout_ref, c_ref, Wg_ref, bg_ref,
                h_ref, cout_ref):
    inp = jnp.concatenate(
        [xin_ref[...], xout_ref[...], hin_ref[...], hout_ref[...]], axis=1)
    g = jnp.dot(inp, Wg_ref[...], preferred_element_type=jnp.float32)
    g = g + bg_ref[...]
    gi = jax.nn.sigmoid(g[:, 0 * DH:1 * DH])
    go = jax.nn.sigmoid(g[:, 1 * DH:2 * DH])
    gf = jax.nn.sigmoid(g[:, 2 * DH:3 * DH])
    gu = jnp.tanh(g[:, 3 * DH:4 * DH])
    cn = gf * c_ref[...] + gi * gu
    cout_ref[...] = cn
    h_ref[...] = go * jnp.tanh(cn)


def _gates(x_in, x_out, h_in, h_out, c, Wg, bg):
    row = pl.BlockSpec((_NBLK, DH), lambda i: (i, 0))
    return pl.pallas_call(
        _gates_body,
        grid=(NP // _NBLK,),
        in_specs=[row, row, row, row, row,
                  pl.BlockSpec((4 * DH, 4 * DH), lambda i: (0, 0)),
                  pl.BlockSpec((1, 4 * DH), lambda i: (0, 0))],
        out_specs=[row, row],
        out_shape=[jax.ShapeDtypeStruct((NP, DH), jnp.float32),
                   jax.ShapeDtypeStruct((NP, DH), jnp.float32)],
    )(x_in, x_out, h_in, h_out, c, Wg, bg)


def kernel(i_token, i_link, i_from, i_to, emb_token, emb_link, W_c, b_c,
           W_l, b_l, W_gi, b_gi, W_go, b_go, W_gf, b_gf, W_gu, b_gu):
    i_token = i_token.astype(jnp.int32)
    i_link = i_link.astype(jnp.int32)
    i_from = i_from.astype(jnp.int32)
    i_to = i_to.astype(jnp.int32)

    tok3 = jnp.pad(i_token, (0, NP - N)).reshape(NW, NP // NW // CH, CH)
    frm3w = i_from.reshape(NW, CW, CH)
    frm3t = i_from.reshape(NS, CT, CH)
    to3t = i_to.reshape(NS, CT, CH)
    sidx4 = jnp.stack([to3t, frm3t])
    gidx4 = jnp.stack([frm3t, to3t])
    sidx5 = sidx4.reshape(NC, NS, CT // BI, BI, CH)
    gidx5 = gidx4.reshape(NC, NS, CT // BI, BI, CH)
    lnk3 = i_link.reshape(E // _EBLK, 1, _EBLK)

    Wl1 = jnp.pad(W_l[:DL], ((0, DW - DL), (0, 0)))
    Wl2 = W_l[DL:]
    embl = jnp.pad(emb_link, ((0, NLBL_PAD - emb_link.shape[0]), (0, DW - DL)))
    bc = b_c.reshape(1, DH)
    bl = b_l.reshape(1, DH)
    Wg = jnp.concatenate([W_gi, W_go, W_gf, W_gu], axis=1)
    bg = jnp.concatenate([b_gi, b_go, b_gf, b_gu]).reshape(1, 4 * DH)
    zeros = jnp.zeros((NP, DH), jnp.float32)

    wemb = _tok_gather(tok3, emb_token)
    nodeL, labelL = _tables(wemb, W_c, bc, Wl1, Wl2, bl, embl)
    s = _edge_gather(frm3w, nodeL)
    u = _edge_tanh(s, lnk3, labelL)
    xs = _x_sums(u, sidx4, zeros)
    x_in, x_out = xs[0], xs[1]

    h, c = _gates(x_in, x_out, zeros, zeros, zeros, Wg, bg)
    for _ in range(NLAYERS - 1):
        hs = _h_sums(h, gidx5, sidx5, zeros)
        h, c = _gates(x_in, x_out, hs[0], hs[1], c, Wg, bg)
    return h[:N]
